# baseline (speedup 1.0000x reference)
import functools

import jax
import jax.numpy as jnp
from jax import lax
from jax.experimental import pallas as pl
from jax.experimental.pallas import tpu as pltpu
from jax.experimental.pallas import tpu_sc as plsc

BATCH = 16384
EMBED_DIM = 32
VOCAB1 = 1000001
VIEW_W = 16
VIEW_ROWS = VOCAB1 * EMBED_DIM // VIEW_W
PER_ID = EMBED_DIM // VIEW_W

_info = plsc.get_sparse_core_info()
_NC, _NS, _NL = _info.num_cores, _info.num_subcores, _info.num_lanes
_NW = _NC * _NS
_B_PER_W = BATCH // _NW
_CHUNK = 128
_N_CHUNK = _B_PER_W * PER_ID // _CHUNK


def _make_gather():
    mesh = plsc.VectorSubcoreMesh(core_axis_name="c", subcore_axis_name="s")

    @functools.partial(
        pl.kernel,
        mesh=mesh,
        out_type=jax.ShapeDtypeStruct((BATCH * PER_ID, VIEW_W), jnp.float32),
        scratch_types=[
            pltpu.VMEM((_B_PER_W,), jnp.int32),
            pltpu.VMEM((_B_PER_W * PER_ID,), jnp.int32),
            pltpu.VMEM((_B_PER_W * PER_ID, VIEW_W), jnp.float32),
            pltpu.SemaphoreType.DMA,
        ],
        compiler_params=pltpu.CompilerParams(
            use_tc_tiling_on_sc=False, needs_layout_passes=False
        ),
    )
    def gather_kernel(idx_hbm, view_hbm, out_hbm, idx_v, idx2_v, rows_v, sem):
        wid = lax.axis_index("s") * _NC + lax.axis_index("c")
        base = wid * _B_PER_W
        pltpu.sync_copy(idx_hbm.at[pl.ds(base, _B_PER_W)], idx_v)
        lanes = lax.iota(jnp.int32, _NL)
        for k in range(_B_PER_W // _NL):
            ids = idx_v[pl.ds(k * _NL, _NL)]
            r0 = ids * jnp.int32(PER_ID)
            pos = lanes * PER_ID + k * (_NL * PER_ID)
            plsc.store_scatter(idx2_v, [pos], r0)
            plsc.store_scatter(idx2_v, [pos + 1], r0 + 1)
        copies = []
        for c in range(_N_CHUNK):
            copies.append(
                pltpu.async_copy(
                    view_hbm.at[idx2_v.at[pl.ds(c * _CHUNK, _CHUNK)]],
                    rows_v.at[pl.ds(c * _CHUNK, _CHUNK)],
                    sem,
                )
            )
        for c in copies:
            c.wait()
        pltpu.sync_copy(
            rows_v,
            out_hbm.at[pl.ds(base * PER_ID, _B_PER_W * PER_ID)],
        )

    return gather_kernel


_gather = _make_gather()


def kernel(user_id, table):
    view = jnp.reshape(table, (VIEW_ROWS, VIEW_W))
    out = _gather(user_id, view)
    return jnp.reshape(out, (BATCH, EMBED_DIM))
